# Initial kernel scaffold; baseline (speedup 1.0000x reference)
import jax
import jax.numpy as jnp
from jax.experimental import pallas as pl


def kernel(positions, edge_index, shifts):
    raise NotImplementedError("write your pallas kernel here")


# speedup vs baseline: 5.2165x; 5.2165x over previous
import functools

import jax
import jax.numpy as jnp
from jax import lax
from jax.experimental import pallas as pl
from jax.experimental.pallas import tpu as pltpu
from jax.experimental.pallas import tpu_sc as plsc

ALPHA = 2.0
N_NODES = 100000
N_EDGES = 6400000

NC = 2
NS = 16
NW = NC * NS

LANES = 16
CHUNK = 2048
ROWS = CHUNK // 128
NROWS_TOTAL = N_EDGES // 128
NCHUNKS = N_EDGES // CHUNK
CH_PER_W = (NCHUNKS + NW - 1) // NW

ZSLICE = 6256
ZLAST = N_NODES - 15 * ZSLICE


def _sc_body(pos_hbm, snd_hbm, rcv_hbm, shifts_hbm, out_hbm,
             sbuf, rbuf, shbuf, psbuf, prbuf, ebuf, zbuf, accum, sem):
    c = lax.axis_index("c")
    s = lax.axis_index("s")
    w = s * NC + c

    def _zb(g, carry):
        zbuf[pl.ds(pl.multiple_of(g * LANES, LANES), LANES)] = jnp.zeros(
            (LANES,), jnp.float32)
        return carry
    lax.fori_loop(0, ZSLICE // LANES, _zb, 0)

    off = pl.multiple_of(s * ZSLICE, 8)

    @pl.when(s < NS - 1)
    def _():
        pltpu.sync_copy(zbuf.at[pl.ds(0, ZSLICE)], accum.at[pl.ds(off, ZSLICE)])

    @pl.when(s == NS - 1)
    def _():
        pltpu.sync_copy(zbuf.at[pl.ds(0, ZLAST)],
                        accum.at[pl.ds(15 * ZSLICE, ZLAST)])

    plsc.subcore_barrier()

    iota = lax.iota(jnp.int32, LANES)
    c0 = jnp.zeros((LANES,), jnp.int32)
    c1 = jnp.full((LANES,), 1, jnp.int32)
    c2 = jnp.full((LANES,), 2, jnp.int32)

    def _chunk(i, carry):
        k = w + i * NW

        @pl.when(k < NCHUNKS)
        def _():
            row0 = k * ROWS
            e0 = k * CHUNK
            pltpu.sync_copy(snd_hbm.at[pl.ds(row0, ROWS)], sbuf)
            pltpu.sync_copy(rcv_hbm.at[pl.ds(row0, ROWS)], rbuf)
            pltpu.sync_copy(shifts_hbm.at[pl.ds(e0, CHUNK)], shbuf)

            descs = []
            for j in range(ROWS):
                descs.append(pltpu.async_copy(
                    pos_hbm.at[sbuf.at[j]],
                    psbuf.at[pl.ds(j * 128, 128)], sem))
                descs.append(pltpu.async_copy(
                    pos_hbm.at[rbuf.at[j]],
                    prbuf.at[pl.ds(j * 128, 128)], sem))
            for d in descs:
                d.wait()

            def _compute(g, carry2):
                rowids = g * LANES + iota
                psx = plsc.load_gather(psbuf, [rowids, c0])
                psy = plsc.load_gather(psbuf, [rowids, c1])
                psz = plsc.load_gather(psbuf, [rowids, c2])
                prx = plsc.load_gather(prbuf, [rowids, c0])
                pry = plsc.load_gather(prbuf, [rowids, c1])
                prz = plsc.load_gather(prbuf, [rowids, c2])
                shx = plsc.load_gather(shbuf, [rowids, c0])
                shy = plsc.load_gather(shbuf, [rowids, c1])
                shz = plsc.load_gather(shbuf, [rowids, c2])
                dx = prx - psx + shx
                dy = pry - psy + shy
                dz = prz - psz + shz
                r2 = dx * dx + dy * dy + dz * dz
                yi = jnp.int32(0x5F3759DF) - (plsc.bitcast(r2, jnp.int32) >> 1)
                y = plsc.bitcast(yi, jnp.float32)
                h = 0.5 * r2
                for _ in range(3):
                    y = y * (1.5 - (h * y) * y)
                r = r2 * y
                e = 0.5 * jnp.exp(-ALPHA * r)
                ebuf[pl.ds(pl.multiple_of(g * LANES, LANES), LANES)] = e
                return carry2

            lax.fori_loop(0, CHUNK // LANES, _compute, 0)

            for j in range(ROWS):
                pltpu.sync_copy(ebuf.at[pl.ds(j * 128, 128)],
                                accum.at[sbuf.at[j]], add=True)
        return carry

    lax.fori_loop(0, CH_PER_W, _chunk, 0)

    plsc.subcore_barrier()

    cbase = c * N_NODES

    @pl.when(s < NS - 1)
    def _():
        pltpu.sync_copy(accum.at[pl.ds(off, ZSLICE)],
                        out_hbm.at[pl.ds(pl.multiple_of(cbase + off, 8),
                                         ZSLICE)])

    @pl.when(s == NS - 1)
    def _():
        pltpu.sync_copy(accum.at[pl.ds(15 * ZSLICE, ZLAST)],
                        out_hbm.at[pl.ds(pl.multiple_of(cbase + 15 * ZSLICE, 8),
                                         ZLAST)])


_sc_call = pl.kernel(
    _sc_body,
    out_type=jax.ShapeDtypeStruct((NC * N_NODES,), jnp.float32),
    mesh=plsc.VectorSubcoreMesh(core_axis_name="c", subcore_axis_name="s",
                                num_cores=NC, num_subcores=NS),
    compiler_params=pltpu.CompilerParams(needs_layout_passes=False,
                                         use_tc_tiling_on_sc=False),
    scratch_types=[
        pltpu.VMEM((ROWS, 128), jnp.int32),
        pltpu.VMEM((ROWS, 128), jnp.int32),
        pltpu.VMEM((CHUNK, 3), jnp.float32),
        pltpu.VMEM((CHUNK, 8), jnp.float32),
        pltpu.VMEM((CHUNK, 8), jnp.float32),
        pltpu.VMEM((CHUNK,), jnp.float32),
        pltpu.VMEM((ZSLICE,), jnp.float32),
        pltpu.VMEM_SHARED((N_NODES,), jnp.float32),
        pltpu.SemaphoreType.DMA,
    ],
)


def _combine_body(part_ref, out_ref):
    out_ref[...] = part_ref[0] + part_ref[1]


_combine = pl.pallas_call(
    _combine_body,
    out_shape=jax.ShapeDtypeStruct((N_NODES,), jnp.float32),
)


def kernel(positions, edge_index, shifts):
    pos8 = jnp.pad(positions, ((0, 0), (0, 5)))
    snd = edge_index[0].reshape(NROWS_TOTAL, 128)
    rcv = edge_index[1].reshape(NROWS_TOTAL, 128)
    partial = _sc_call(pos8, snd, rcv, shifts)
    return _combine(partial.reshape(NC, N_NODES))


# speedup vs baseline: 6.2522x; 1.1985x over previous
import jax
import jax.numpy as jnp
from jax import lax
from jax.experimental import pallas as pl
from jax.experimental.pallas import tpu as pltpu
from jax.experimental.pallas import tpu_sc as plsc

ALPHA = 2.0
N_NODES = 100000
N_EDGES = 6400000

NC = 2
NS = 16
NW = NC * NS

LANES = 16
CHUNK = 2048
NCHUNKS = N_EDGES // CHUNK
CH_PER_W = (NCHUNKS + NW - 1) // NW

ZSLICE = 6256
ZLAST = N_NODES - 15 * ZSLICE


def _sc_body(pos_hbm, snd_hbm, rcv_hbm, shifts_hbm, out_hbm,
             sbuf, rbuf, shbuf, psbuf, prbuf, ebuf, zbuf, accum, sem):
    c = lax.axis_index("c")
    s = lax.axis_index("s")
    w = s * NC + c

    def _zb(g, carry):
        zbuf[pl.ds(pl.multiple_of(g * LANES, LANES), LANES)] = jnp.zeros(
            (LANES,), jnp.float32)
        return carry
    lax.fori_loop(0, ZSLICE // LANES, _zb, 0)

    off = pl.multiple_of(s * ZSLICE, 8)

    @pl.when(s < NS - 1)
    def _():
        pltpu.sync_copy(zbuf.at[pl.ds(0, ZSLICE)], accum.at[pl.ds(off, ZSLICE)])

    @pl.when(s == NS - 1)
    def _():
        pltpu.sync_copy(zbuf.at[pl.ds(0, ZLAST)],
                        accum.at[pl.ds(15 * ZSLICE, ZLAST)])

    plsc.subcore_barrier()

    iota = lax.iota(jnp.int32, LANES)
    c0 = jnp.zeros((LANES,), jnp.int32)
    c1 = jnp.full((LANES,), 1, jnp.int32)
    c2 = jnp.full((LANES,), 2, jnp.int32)
    iota3 = iota * 3

    def _chunk(i, carry):
        k = w + i * NW

        @pl.when(k < NCHUNKS)
        def _():
            e0 = pl.multiple_of(k * CHUNK, 8)
            pltpu.sync_copy(snd_hbm.at[pl.ds(e0, CHUNK)], sbuf)
            pltpu.sync_copy(rcv_hbm.at[pl.ds(e0, CHUNK)], rbuf)
            pltpu.sync_copy(
                shifts_hbm.at[pl.ds(pl.multiple_of(k * (3 * CHUNK), 8),
                                    3 * CHUNK)], shbuf)

            ds_ = pltpu.async_copy(pos_hbm.at[sbuf], psbuf, sem)
            dr_ = pltpu.async_copy(pos_hbm.at[rbuf], prbuf, sem)
            ds_.wait()
            dr_.wait()

            def _compute(g, carry2):
                rowids = g * LANES + iota
                sh0 = g * (3 * LANES) + iota3
                psx = plsc.load_gather(psbuf, [rowids, c0])
                psy = plsc.load_gather(psbuf, [rowids, c1])
                psz = plsc.load_gather(psbuf, [rowids, c2])
                prx = plsc.load_gather(prbuf, [rowids, c0])
                pry = plsc.load_gather(prbuf, [rowids, c1])
                prz = plsc.load_gather(prbuf, [rowids, c2])
                shx = plsc.load_gather(shbuf, [sh0])
                shy = plsc.load_gather(shbuf, [sh0 + 1])
                shz = plsc.load_gather(shbuf, [sh0 + 2])
                dx = prx - psx + shx
                dy = pry - psy + shy
                dz = prz - psz + shz
                r2 = dx * dx + dy * dy + dz * dz
                yi = jnp.int32(0x5F3759DF) - (plsc.bitcast(r2, jnp.int32) >> 1)
                y = plsc.bitcast(yi, jnp.float32)
                h = 0.5 * r2
                for _ in range(3):
                    y = y * (1.5 - (h * y) * y)
                r = r2 * y
                e = 0.5 * jnp.exp(-ALPHA * r)
                ebuf[pl.ds(pl.multiple_of(g * LANES, LANES), LANES)] = e
                return carry2

            lax.fori_loop(0, CHUNK // LANES, _compute, 0)

            pltpu.sync_copy(ebuf, accum.at[sbuf], add=True)
        return carry

    lax.fori_loop(0, CH_PER_W, _chunk, 0)

    plsc.subcore_barrier()

    cbase = c * N_NODES

    @pl.when(s < NS - 1)
    def _():
        pltpu.sync_copy(accum.at[pl.ds(off, ZSLICE)],
                        out_hbm.at[pl.ds(pl.multiple_of(cbase + off, 8),
                                         ZSLICE)])

    @pl.when(s == NS - 1)
    def _():
        pltpu.sync_copy(accum.at[pl.ds(15 * ZSLICE, ZLAST)],
                        out_hbm.at[pl.ds(pl.multiple_of(cbase + 15 * ZSLICE, 8),
                                         ZLAST)])


_sc_call = pl.kernel(
    _sc_body,
    out_type=jax.ShapeDtypeStruct((NC * N_NODES,), jnp.float32),
    mesh=plsc.VectorSubcoreMesh(core_axis_name="c", subcore_axis_name="s",
                                num_cores=NC, num_subcores=NS),
    compiler_params=pltpu.CompilerParams(needs_layout_passes=False,
                                         use_tc_tiling_on_sc=False),
    scratch_types=[
        pltpu.VMEM((CHUNK,), jnp.int32),
        pltpu.VMEM((CHUNK,), jnp.int32),
        pltpu.VMEM((3 * CHUNK,), jnp.float32),
        pltpu.VMEM((CHUNK, 8), jnp.float32),
        pltpu.VMEM((CHUNK, 8), jnp.float32),
        pltpu.VMEM((CHUNK,), jnp.float32),
        pltpu.VMEM((ZSLICE,), jnp.float32),
        pltpu.VMEM_SHARED((N_NODES,), jnp.float32),
        pltpu.SemaphoreType.DMA,
    ],
)


def _combine_body(part_ref, out_ref):
    out_ref[...] = part_ref[0] + part_ref[1]


_combine = pl.pallas_call(
    _combine_body,
    out_shape=jax.ShapeDtypeStruct((N_NODES,), jnp.float32),
)


def kernel(positions, edge_index, shifts):
    pos8 = jnp.pad(positions, ((0, 0), (0, 5)))
    snd = edge_index[0]
    rcv = edge_index[1]
    shifts_flat = shifts.reshape(3 * N_EDGES)
    partial = _sc_call(pos8, snd, rcv, shifts_flat)
    return _combine(partial.reshape(NC, N_NODES))


# speedup vs baseline: 35.8539x; 5.7346x over previous
import jax
import jax.numpy as jnp
from jax import lax
from jax.experimental import pallas as pl
from jax.experimental.pallas import tpu as pltpu
from jax.experimental.pallas import tpu_sc as plsc

ALPHA = 2.0
N_NODES = 100000
N_EDGES = 6400000

NC = 2
NS = 16
NW = NC * NS

LANES = 16
CHUNK = 2048
NCHUNKS = N_EDGES // CHUNK
CH_PER_W = (NCHUNKS + NW - 1) // NW

ZSLICE = 6256
ZLAST = N_NODES - 15 * ZSLICE


def _sc_body(pos_hbm, snd_hbm, rcv_hbm, sx_hbm, sy_hbm, sz_hbm, out_hbm,
             sbuf, rbuf, sxbuf, sybuf, szbuf, psbuf, prbuf, ebuf, zbuf,
             accum, sem):
    c = lax.axis_index("c")
    s = lax.axis_index("s")
    w = s * NC + c

    def _zb(g, carry):
        zbuf[pl.ds(pl.multiple_of(g * LANES, LANES), LANES)] = jnp.zeros(
            (LANES,), jnp.float32)
        return carry
    lax.fori_loop(0, ZSLICE // LANES, _zb, 0)

    off = pl.multiple_of(s * ZSLICE, 8)

    @pl.when(s < NS - 1)
    def _():
        pltpu.sync_copy(zbuf.at[pl.ds(0, ZSLICE)], accum.at[pl.ds(off, ZSLICE)])

    @pl.when(s == NS - 1)
    def _():
        pltpu.sync_copy(zbuf.at[pl.ds(0, ZLAST)],
                        accum.at[pl.ds(15 * ZSLICE, ZLAST)])

    plsc.subcore_barrier()

    iota = lax.iota(jnp.int32, LANES)
    c0 = jnp.zeros((LANES,), jnp.int32)
    c1 = jnp.full((LANES,), 1, jnp.int32)
    c2 = jnp.full((LANES,), 2, jnp.int32)

    def _chunk(i, carry):
        k = w + i * NW

        @pl.when(k < NCHUNKS)
        def _():
            e0 = pl.multiple_of(k * CHUNK, 8)
            pltpu.sync_copy(snd_hbm.at[pl.ds(e0, CHUNK)], sbuf)
            pltpu.sync_copy(rcv_hbm.at[pl.ds(e0, CHUNK)], rbuf)
            pltpu.sync_copy(sx_hbm.at[pl.ds(e0, CHUNK)], sxbuf)
            pltpu.sync_copy(sy_hbm.at[pl.ds(e0, CHUNK)], sybuf)
            pltpu.sync_copy(sz_hbm.at[pl.ds(e0, CHUNK)], szbuf)

            ds_ = pltpu.async_copy(pos_hbm.at[sbuf], psbuf, sem)
            dr_ = pltpu.async_copy(pos_hbm.at[rbuf], prbuf, sem)
            ds_.wait()
            dr_.wait()

            def _compute(g, carry2):
                rowids = g * LANES + iota
                eoff = pl.ds(pl.multiple_of(g * LANES, LANES), LANES)
                psx = plsc.load_gather(psbuf, [rowids, c0])
                psy = plsc.load_gather(psbuf, [rowids, c1])
                psz = plsc.load_gather(psbuf, [rowids, c2])
                prx = plsc.load_gather(prbuf, [rowids, c0])
                pry = plsc.load_gather(prbuf, [rowids, c1])
                prz = plsc.load_gather(prbuf, [rowids, c2])
                shx = sxbuf[eoff]
                shy = sybuf[eoff]
                shz = szbuf[eoff]
                dx = prx - psx + shx
                dy = pry - psy + shy
                dz = prz - psz + shz
                r2 = dx * dx + dy * dy + dz * dz
                yi = jnp.int32(0x5F3759DF) - (plsc.bitcast(r2, jnp.int32) >> 1)
                y = plsc.bitcast(yi, jnp.float32)
                h = 0.5 * r2
                for _ in range(3):
                    y = y * (1.5 - (h * y) * y)
                r = r2 * y
                e = 0.5 * jnp.exp(-ALPHA * r)
                ebuf[eoff] = e
                return carry2

            lax.fori_loop(0, CHUNK // LANES, _compute, 0)

            pltpu.sync_copy(ebuf, accum.at[sbuf], add=True)
        return carry

    lax.fori_loop(0, CH_PER_W, _chunk, 0)

    plsc.subcore_barrier()

    cbase = c * N_NODES

    @pl.when(s < NS - 1)
    def _():
        pltpu.sync_copy(accum.at[pl.ds(off, ZSLICE)],
                        out_hbm.at[pl.ds(pl.multiple_of(cbase + off, 8),
                                         ZSLICE)])

    @pl.when(s == NS - 1)
    def _():
        pltpu.sync_copy(accum.at[pl.ds(15 * ZSLICE, ZLAST)],
                        out_hbm.at[pl.ds(pl.multiple_of(cbase + 15 * ZSLICE, 8),
                                         ZLAST)])


_sc_call = pl.kernel(
    _sc_body,
    out_type=jax.ShapeDtypeStruct((NC * N_NODES,), jnp.float32),
    mesh=plsc.VectorSubcoreMesh(core_axis_name="c", subcore_axis_name="s",
                                num_cores=NC, num_subcores=NS),
    compiler_params=pltpu.CompilerParams(needs_layout_passes=False,
                                         use_tc_tiling_on_sc=False),
    scratch_types=[
        pltpu.VMEM((CHUNK,), jnp.int32),
        pltpu.VMEM((CHUNK,), jnp.int32),
        pltpu.VMEM((CHUNK,), jnp.float32),
        pltpu.VMEM((CHUNK,), jnp.float32),
        pltpu.VMEM((CHUNK,), jnp.float32),
        pltpu.VMEM((CHUNK, 8), jnp.float32),
        pltpu.VMEM((CHUNK, 8), jnp.float32),
        pltpu.VMEM((CHUNK,), jnp.float32),
        pltpu.VMEM((ZSLICE,), jnp.float32),
        pltpu.VMEM_SHARED((N_NODES,), jnp.float32),
        pltpu.SemaphoreType.DMA,
    ],
)


def _combine_body(part_ref, out_ref):
    out_ref[...] = part_ref[0] + part_ref[1]


_combine = pl.pallas_call(
    _combine_body,
    out_shape=jax.ShapeDtypeStruct((N_NODES,), jnp.float32),
)


def kernel(positions, edge_index, shifts):
    pos8 = jnp.pad(positions, ((0, 0), (0, 5)))
    snd = edge_index[0]
    rcv = edge_index[1]
    partial = _sc_call(pos8, snd, rcv, shifts[:, 0], shifts[:, 1],
                       shifts[:, 2])
    return _combine(partial.reshape(NC, N_NODES))


# speedup vs baseline: 71.5354x; 1.9952x over previous
import jax
import jax.numpy as jnp
from jax import lax
from jax.experimental import pallas as pl
from jax.experimental.pallas import tpu as pltpu
from jax.experimental.pallas import tpu_sc as plsc

ALPHA = 2.0
N_NODES = 100000
N_EDGES = 6400000

NC = 2
NS = 16
NW = NC * NS

LANES = 16
CHUNK = 2048
NCHUNKS = N_EDGES // CHUNK
CH_PER_W = (NCHUNKS + NW - 1) // NW
OUTER = (CH_PER_W + 2 + 3) // 4

ZSLICE = 6256
ZLAST = N_NODES - 15 * ZSLICE


def _sc_body(pos_hbm, snd_hbm, rcv_hbm, sx_hbm, sy_hbm, sz_hbm, out_hbm,
             sbufs, rbufs, sxbs, sybs, szbs, psbs, prbs, ebs, zbuf, accum,
             sem_lin, sem_g, sem_s):
    c = lax.axis_index("c")
    s = lax.axis_index("s")
    w = s * NC + c

    def _zb(g, carry):
        zbuf[pl.ds(pl.multiple_of(g * LANES, LANES), LANES)] = jnp.zeros(
            (LANES,), jnp.float32)
        return carry
    lax.fori_loop(0, ZSLICE // LANES, _zb, 0)

    off = pl.multiple_of(s * ZSLICE, 8)

    @pl.when(s < NS - 1)
    def _():
        pltpu.sync_copy(zbuf.at[pl.ds(0, ZSLICE)], accum.at[pl.ds(off, ZSLICE)])

    @pl.when(s == NS - 1)
    def _():
        pltpu.sync_copy(zbuf.at[pl.ds(0, ZLAST)],
                        accum.at[pl.ds(15 * ZSLICE, ZLAST)])

    plsc.subcore_barrier()

    iota = lax.iota(jnp.int32, LANES)
    c0 = jnp.zeros((LANES,), jnp.int32)
    c1 = jnp.full((LANES,), 1, jnp.int32)
    c2 = jnp.full((LANES,), 2, jnp.int32)

    def chunk_of(i):
        return w + i * NW

    def fire_lin(i, u):
        u = u % 4
        b = u % 2
        k = chunk_of(i)

        @pl.when(k < NCHUNKS)
        def _():
            e0 = pl.multiple_of(k * CHUNK, 8)
            sl = pl.ds(e0, CHUNK)
            pltpu.async_copy(snd_hbm.at[sl], sbufs[u], sem_lin[b])
            pltpu.async_copy(rcv_hbm.at[sl], rbufs[b], sem_lin[b])
            pltpu.async_copy(sx_hbm.at[sl], sxbs[b], sem_lin[b])
            pltpu.async_copy(sy_hbm.at[sl], sybs[b], sem_lin[b])
            pltpu.async_copy(sz_hbm.at[sl], szbs[b], sem_lin[b])

    def wait_lin(i, u):
        u = u % 4
        b = u % 2
        k = chunk_of(i)

        @pl.when(k < NCHUNKS)
        def _():
            sl = pl.ds(0, CHUNK)
            pltpu.make_async_copy(snd_hbm.at[sl], sbufs[u], sem_lin[b]).wait()
            pltpu.make_async_copy(rcv_hbm.at[sl], rbufs[b], sem_lin[b]).wait()
            pltpu.make_async_copy(sx_hbm.at[sl], sxbs[b], sem_lin[b]).wait()
            pltpu.make_async_copy(sy_hbm.at[sl], sybs[b], sem_lin[b]).wait()
            pltpu.make_async_copy(sz_hbm.at[sl], szbs[b], sem_lin[b]).wait()

    def fire_gath(i, u):
        u = u % 4
        b = u % 2
        k = chunk_of(i)

        @pl.when(k < NCHUNKS)
        def _():
            pltpu.async_copy(pos_hbm.at[sbufs[u]], psbs[b], sem_g[b])
            pltpu.async_copy(pos_hbm.at[rbufs[b]], prbs[b], sem_g[b])

    def wait_gath(i, u):
        u = u % 4
        b = u % 2
        k = chunk_of(i)

        @pl.when(k < NCHUNKS)
        def _():
            dummy = pos_hbm.at[pl.ds(0, CHUNK)]
            pltpu.make_async_copy(dummy, psbs[b], sem_g[b]).wait()
            pltpu.make_async_copy(dummy, prbs[b], sem_g[b]).wait()

    def fire_sct(i, u):
        u = u % 4
        b = u % 2
        k = chunk_of(i)

        @pl.when(k < NCHUNKS)
        def _():
            pltpu.async_copy(ebs[b], accum.at[sbufs[u]], sem_s[b], add=True)

    def wait_sct(i, u):
        b = (u % 4) % 2
        k = chunk_of(i)

        @pl.when((i >= 0) & (k < NCHUNKS))
        def _():
            pltpu.make_async_copy(sx_hbm.at[pl.ds(0, CHUNK)], ebs[b],
                                  sem_s[b]).wait()

    def compute(i, u):
        u = u % 4
        b = u % 2
        k = chunk_of(i)
        psb, prb = psbs[b], prbs[b]
        sxb, syb, szb = sxbs[b], sybs[b], szbs[b]
        eb = ebs[b]

        @pl.when(k < NCHUNKS)
        def _():
            @plsc.parallel_loop(0, CHUNK // LANES, 1, unroll=2)
            def _(g):
                rowids = g * LANES + iota
                eoff = pl.ds(pl.multiple_of(g * LANES, LANES), LANES)
                psx = plsc.load_gather(psb, [rowids, c0])
                psy = plsc.load_gather(psb, [rowids, c1])
                psz = plsc.load_gather(psb, [rowids, c2])
                prx = plsc.load_gather(prb, [rowids, c0])
                pry = plsc.load_gather(prb, [rowids, c1])
                prz = plsc.load_gather(prb, [rowids, c2])
                dx = prx - psx + sxb[eoff]
                dy = pry - psy + syb[eoff]
                dz = prz - psz + szb[eoff]
                r2 = dx * dx + dy * dy + dz * dz
                yi = jnp.int32(0x5F3759DF) - (plsc.bitcast(r2, jnp.int32) >> 1)
                y = plsc.bitcast(yi, jnp.float32)
                h = 0.5 * r2
                for _ in range(3):
                    y = y * (1.5 - (h * y) * y)
                r = r2 * y
                eb[eoff] = 0.5 * jnp.exp(-ALPHA * r)

    fire_lin(0, 0)
    fire_lin(1, 1)
    wait_lin(0, 0)
    fire_gath(0, 0)

    def _outer(i2, carry):
        for u in range(4):
            i = i2 * 4 + u
            wait_gath(i, u)
            wait_lin(i + 1, u + 1)
            fire_gath(i + 1, u + 1)
            wait_sct(i - 2, u)
            compute(i, u)
            fire_sct(i, u)
            fire_lin(i + 2, u + 2)
        return carry

    lax.fori_loop(0, OUTER, _outer, 0)

    plsc.subcore_barrier()

    cbase = c * N_NODES

    @pl.when(s < NS - 1)
    def _():
        pltpu.sync_copy(accum.at[pl.ds(off, ZSLICE)],
                        out_hbm.at[pl.ds(pl.multiple_of(cbase + off, 8),
                                         ZSLICE)])

    @pl.when(s == NS - 1)
    def _():
        pltpu.sync_copy(accum.at[pl.ds(15 * ZSLICE, ZLAST)],
                        out_hbm.at[pl.ds(pl.multiple_of(cbase + 15 * ZSLICE, 8),
                                         ZLAST)])


_sc_call = pl.kernel(
    _sc_body,
    out_type=jax.ShapeDtypeStruct((NC * N_NODES,), jnp.float32),
    mesh=plsc.VectorSubcoreMesh(core_axis_name="c", subcore_axis_name="s",
                                num_cores=NC, num_subcores=NS),
    compiler_params=pltpu.CompilerParams(needs_layout_passes=False,
                                         use_tc_tiling_on_sc=False),
    scratch_types=[
        tuple(pltpu.VMEM((CHUNK,), jnp.int32) for _ in range(4)),
        tuple(pltpu.VMEM((CHUNK,), jnp.int32) for _ in range(2)),
        tuple(pltpu.VMEM((CHUNK,), jnp.float32) for _ in range(2)),
        tuple(pltpu.VMEM((CHUNK,), jnp.float32) for _ in range(2)),
        tuple(pltpu.VMEM((CHUNK,), jnp.float32) for _ in range(2)),
        tuple(pltpu.VMEM((CHUNK, 8), jnp.float32) for _ in range(2)),
        tuple(pltpu.VMEM((CHUNK, 8), jnp.float32) for _ in range(2)),
        tuple(pltpu.VMEM((CHUNK,), jnp.float32) for _ in range(2)),
        pltpu.VMEM((ZSLICE,), jnp.float32),
        pltpu.VMEM_SHARED((N_NODES,), jnp.float32),
        tuple(pltpu.SemaphoreType.DMA for _ in range(2)),
        tuple(pltpu.SemaphoreType.DMA for _ in range(2)),
        tuple(pltpu.SemaphoreType.DMA for _ in range(2)),
    ],
)


def _combine_body(part_ref, out_ref):
    out_ref[...] = part_ref[0] + part_ref[1]


_combine = pl.pallas_call(
    _combine_body,
    out_shape=jax.ShapeDtypeStruct((N_NODES,), jnp.float32),
)


def kernel(positions, edge_index, shifts):
    pos8 = jnp.pad(positions, ((0, 0), (0, 5)))
    snd = edge_index[0]
    rcv = edge_index[1]
    partial = _sc_call(pos8, snd, rcv, shifts[:, 0], shifts[:, 1],
                       shifts[:, 2])
    return _combine(partial.reshape(NC, N_NODES))


# speedup vs baseline: 75.2467x; 1.0519x over previous
import jax
import jax.numpy as jnp
from jax import lax
from jax.experimental import pallas as pl
from jax.experimental.pallas import tpu as pltpu
from jax.experimental.pallas import tpu_sc as plsc

ALPHA = 2.0
N_NODES = 100000
N_EDGES = 6400000

NC = 2
NS = 16
NW = NC * NS

LANES = 16
CHUNK = 2048
NCHUNKS = N_EDGES // CHUNK
CH_PER_W = (NCHUNKS + NW - 1) // NW
OUTER = (CH_PER_W + 2 + 3) // 4

ZSLICE = 6256
ZLAST = N_NODES - 15 * ZSLICE


def _sc_body(pos_hbm, ei_hbm, sx_hbm, sy_hbm, sz_hbm, out_hbm,
             ibufs, sbufs, rbufs, sxbs, sybs, szbs, psbs, prbs, ebs, zbuf,
             accum, sem_lin, sem_g, sem_s):
    c = lax.axis_index("c")
    s = lax.axis_index("s")
    w = s * NC + c

    def _zb(g, carry):
        zbuf[pl.ds(pl.multiple_of(g * LANES, LANES), LANES)] = jnp.zeros(
            (LANES,), jnp.float32)
        return carry
    lax.fori_loop(0, ZSLICE // LANES, _zb, 0)

    off = pl.multiple_of(s * ZSLICE, 8)

    @pl.when(s < NS - 1)
    def _():
        pltpu.sync_copy(zbuf.at[pl.ds(0, ZSLICE)], accum.at[pl.ds(off, ZSLICE)])

    @pl.when(s == NS - 1)
    def _():
        pltpu.sync_copy(zbuf.at[pl.ds(0, ZLAST)],
                        accum.at[pl.ds(15 * ZSLICE, ZLAST)])

    plsc.subcore_barrier()

    iota = lax.iota(jnp.int32, LANES)
    c0 = jnp.zeros((LANES,), jnp.int32)
    c1 = jnp.full((LANES,), 1, jnp.int32)
    c2 = jnp.full((LANES,), 2, jnp.int32)

    def chunk_of(i):
        return w + i * NW

    ROWS = CHUNK // 128

    def fire_lin(i, u):
        u = u % 4
        b = u % 2
        k = chunk_of(i)

        @pl.when(k < NCHUNKS)
        def _():
            e0 = pl.multiple_of(k * CHUNK, 8)
            sl = pl.ds(e0, CHUNK)
            pltpu.async_copy(ei_hbm.at[pl.ds(k * ROWS, ROWS)], ibufs[u],
                             sem_lin[b])
            pltpu.async_copy(sx_hbm.at[sl], sxbs[b], sem_lin[b])
            pltpu.async_copy(sy_hbm.at[sl], sybs[b], sem_lin[b])
            pltpu.async_copy(sz_hbm.at[sl], szbs[b], sem_lin[b])

    def wait_lin(i, u):
        u = u % 4
        b = u % 2
        k = chunk_of(i)

        @pl.when(k < NCHUNKS)
        def _():
            sl = pl.ds(0, CHUNK)
            pltpu.make_async_copy(ei_hbm.at[pl.ds(0, ROWS)], ibufs[u],
                                  sem_lin[b]).wait()
            pltpu.make_async_copy(sx_hbm.at[sl], sxbs[b], sem_lin[b]).wait()
            pltpu.make_async_copy(sy_hbm.at[sl], sybs[b], sem_lin[b]).wait()
            pltpu.make_async_copy(sz_hbm.at[sl], szbs[b], sem_lin[b]).wait()

    def deinterleave(i, u):
        u = u % 4
        b = u % 2
        k = chunk_of(i)
        ib, sb, rb = ibufs[u], sbufs[u], rbufs[b]

        @pl.when(k < NCHUNKS)
        def _():
            @plsc.parallel_loop(0, ROWS, 1)
            def _(j):
                for l in range(8):
                    o = pl.ds(pl.multiple_of(j * 128 + l * 16, 16), LANES)
                    sb[o] = ib[j, 0, pl.ds(l * 16, LANES)]
                    rb[o] = ib[j, 1, pl.ds(l * 16, LANES)]

    def fire_gath(i, u):
        u = u % 4
        b = u % 2
        k = chunk_of(i)

        @pl.when(k < NCHUNKS)
        def _():
            pltpu.async_copy(pos_hbm.at[sbufs[u]], psbs[b], sem_g[b])
            pltpu.async_copy(pos_hbm.at[rbufs[b]], prbs[b], sem_g[b])

    def wait_gath(i, u):
        u = u % 4
        b = u % 2
        k = chunk_of(i)

        @pl.when(k < NCHUNKS)
        def _():
            dummy = pos_hbm.at[pl.ds(0, CHUNK)]
            pltpu.make_async_copy(dummy, psbs[b], sem_g[b]).wait()
            pltpu.make_async_copy(dummy, prbs[b], sem_g[b]).wait()

    def fire_sct(i, u):
        u = u % 4
        b = u % 2
        k = chunk_of(i)

        @pl.when(k < NCHUNKS)
        def _():
            pltpu.async_copy(ebs[b], accum.at[sbufs[u]], sem_s[b], add=True)

    def wait_sct(i, u):
        b = (u % 4) % 2
        k = chunk_of(i)

        @pl.when((i >= 0) & (k < NCHUNKS))
        def _():
            pltpu.make_async_copy(sx_hbm.at[pl.ds(0, CHUNK)], ebs[b],
                                  sem_s[b]).wait()

    def compute(i, u):
        u = u % 4
        b = u % 2
        k = chunk_of(i)
        psb, prb = psbs[b], prbs[b]
        sxb, syb, szb = sxbs[b], sybs[b], szbs[b]
        eb = ebs[b]

        @pl.when(k < NCHUNKS)
        def _():
            @plsc.parallel_loop(0, CHUNK // LANES, 1, unroll=2)
            def _(g):
                rowids = g * LANES + iota
                eoff = pl.ds(pl.multiple_of(g * LANES, LANES), LANES)
                psx = plsc.load_gather(psb, [rowids, c0])
                psy = plsc.load_gather(psb, [rowids, c1])
                psz = plsc.load_gather(psb, [rowids, c2])
                prx = plsc.load_gather(prb, [rowids, c0])
                pry = plsc.load_gather(prb, [rowids, c1])
                prz = plsc.load_gather(prb, [rowids, c2])
                dx = prx - psx + sxb[eoff]
                dy = pry - psy + syb[eoff]
                dz = prz - psz + szb[eoff]
                r2 = dx * dx + dy * dy + dz * dz
                yi = jnp.int32(0x5F3759DF) - (plsc.bitcast(r2, jnp.int32) >> 1)
                y = plsc.bitcast(yi, jnp.float32)
                h = 0.5 * r2
                for _ in range(3):
                    y = y * (1.5 - (h * y) * y)
                r = r2 * y
                eb[eoff] = 0.5 * jnp.exp(-ALPHA * r)

    fire_lin(0, 0)
    fire_lin(1, 1)
    wait_lin(0, 0)
    deinterleave(0, 0)
    fire_gath(0, 0)

    def _outer(i2, carry):
        for u in range(4):
            i = i2 * 4 + u
            wait_gath(i, u)
            wait_lin(i + 1, u + 1)
            deinterleave(i + 1, u + 1)
            fire_gath(i + 1, u + 1)
            wait_sct(i - 2, u)
            compute(i, u)
            fire_sct(i, u)
            fire_lin(i + 2, u + 2)
        return carry

    lax.fori_loop(0, OUTER, _outer, 0)

    plsc.subcore_barrier()

    cbase = c * N_NODES

    @pl.when(s < NS - 1)
    def _():
        pltpu.sync_copy(accum.at[pl.ds(off, ZSLICE)],
                        out_hbm.at[pl.ds(pl.multiple_of(cbase + off, 8),
                                         ZSLICE)])

    @pl.when(s == NS - 1)
    def _():
        pltpu.sync_copy(accum.at[pl.ds(15 * ZSLICE, ZLAST)],
                        out_hbm.at[pl.ds(pl.multiple_of(cbase + 15 * ZSLICE, 8),
                                         ZLAST)])


_sc_call = pl.kernel(
    _sc_body,
    out_type=jax.ShapeDtypeStruct((NC * N_NODES,), jnp.float32),
    mesh=plsc.VectorSubcoreMesh(core_axis_name="c", subcore_axis_name="s",
                                num_cores=NC, num_subcores=NS),
    compiler_params=pltpu.CompilerParams(needs_layout_passes=False,
                                         use_tc_tiling_on_sc=False),
    scratch_types=[
        tuple(pltpu.VMEM((CHUNK // 128, 2, 128), jnp.int32)
              for _ in range(4)),
        tuple(pltpu.VMEM((CHUNK,), jnp.int32) for _ in range(4)),
        tuple(pltpu.VMEM((CHUNK,), jnp.int32) for _ in range(2)),
        tuple(pltpu.VMEM((CHUNK,), jnp.float32) for _ in range(2)),
        tuple(pltpu.VMEM((CHUNK,), jnp.float32) for _ in range(2)),
        tuple(pltpu.VMEM((CHUNK,), jnp.float32) for _ in range(2)),
        tuple(pltpu.VMEM((CHUNK, 8), jnp.float32) for _ in range(2)),
        tuple(pltpu.VMEM((CHUNK, 8), jnp.float32) for _ in range(2)),
        tuple(pltpu.VMEM((CHUNK,), jnp.float32) for _ in range(2)),
        pltpu.VMEM((ZSLICE,), jnp.float32),
        pltpu.VMEM_SHARED((N_NODES,), jnp.float32),
        tuple(pltpu.SemaphoreType.DMA for _ in range(2)),
        tuple(pltpu.SemaphoreType.DMA for _ in range(2)),
        tuple(pltpu.SemaphoreType.DMA for _ in range(2)),
    ],
)


def _combine_body(part_ref, out_ref):
    out_ref[...] = part_ref[0] + part_ref[1]


_combine = pl.pallas_call(
    _combine_body,
    out_shape=jax.ShapeDtypeStruct((N_NODES,), jnp.float32),
)


def kernel(positions, edge_index, shifts):
    pos8 = jnp.pad(positions, ((0, 0), (0, 5)))
    ei3 = edge_index.reshape(2, N_EDGES // 128, 128).transpose(1, 0, 2)
    partial = _sc_call(pos8, ei3, shifts[:, 0], shifts[:, 1], shifts[:, 2])
    return _combine(partial.reshape(NC, N_NODES))


# speedup vs baseline: 78.0043x; 1.0366x over previous
import jax
import jax.numpy as jnp
from jax import lax
from jax.experimental import pallas as pl
from jax.experimental.pallas import tpu as pltpu
from jax.experimental.pallas import tpu_sc as plsc

ALPHA = 2.0
N_NODES = 100000
N_EDGES = 6400000

NC = 2
NS = 16
NW = NC * NS

LANES = 16
CHUNK = 2048
NCHUNKS = N_EDGES // CHUNK
CH_PER_W = (NCHUNKS + NW - 1) // NW
OUTER = (CH_PER_W + 2 + 3) // 4

ZSLICE = 6256
ZLAST = N_NODES - 15 * ZSLICE


def _sc_body(pos_hbm, ei_hbm, sx_hbm, sy_hbm, sz_hbm, out_hbm,
             ibufs, sbufs, rbufs, sxbs, sybs, szbs, psbs, prbs, ebs, zbuf,
             accum, sem_lin, sem_g, sem_s):
    c = lax.axis_index("c")
    s = lax.axis_index("s")
    w = s * NC + c

    def _zb(g, carry):
        zbuf[pl.ds(pl.multiple_of(g * LANES, LANES), LANES)] = jnp.zeros(
            (LANES,), jnp.float32)
        return carry
    lax.fori_loop(0, ZSLICE // LANES, _zb, 0)

    off = pl.multiple_of(s * ZSLICE, 8)

    @pl.when(s < NS - 1)
    def _():
        pltpu.sync_copy(zbuf.at[pl.ds(0, ZSLICE)], accum.at[pl.ds(off, ZSLICE)])

    @pl.when(s == NS - 1)
    def _():
        pltpu.sync_copy(zbuf.at[pl.ds(0, ZLAST)],
                        accum.at[pl.ds(15 * ZSLICE, ZLAST)])

    plsc.subcore_barrier()

    iota = lax.iota(jnp.int32, LANES)
    c0 = jnp.zeros((LANES,), jnp.int32)
    c1 = jnp.full((LANES,), 1, jnp.int32)
    c2 = jnp.full((LANES,), 2, jnp.int32)

    def chunk_of(i):
        return w + i * NW

    ROWS = CHUNK // 128

    def fire_lin(i, u):
        u = u % 4
        b = u % 2
        k = chunk_of(i)

        @pl.when(k < NCHUNKS)
        def _():
            e0 = pl.multiple_of(k * CHUNK, 8)
            sl = pl.ds(e0, CHUNK)
            pltpu.async_copy(ei_hbm.at[pl.ds(k * ROWS, ROWS)], ibufs[u],
                             sem_lin[b])
            pltpu.async_copy(sx_hbm.at[sl], sxbs[b], sem_lin[b])
            pltpu.async_copy(sy_hbm.at[sl], sybs[b], sem_lin[b])
            pltpu.async_copy(sz_hbm.at[sl], szbs[b], sem_lin[b])

    def wait_lin(i, u):
        u = u % 4
        b = u % 2
        k = chunk_of(i)

        @pl.when(k < NCHUNKS)
        def _():
            sl = pl.ds(0, CHUNK)
            pltpu.make_async_copy(ei_hbm.at[pl.ds(0, ROWS)], ibufs[u],
                                  sem_lin[b]).wait()
            pltpu.make_async_copy(sx_hbm.at[sl], sxbs[b], sem_lin[b]).wait()
            pltpu.make_async_copy(sy_hbm.at[sl], sybs[b], sem_lin[b]).wait()
            pltpu.make_async_copy(sz_hbm.at[sl], szbs[b], sem_lin[b]).wait()

    def deinterleave(i, u):
        u = u % 4
        b = u % 2
        k = chunk_of(i)
        ib, sb, rb = ibufs[u], sbufs[u], rbufs[b]

        @pl.when(k < NCHUNKS)
        def _():
            @plsc.parallel_loop(0, ROWS, 1)
            def _(j):
                for l in range(8):
                    o = pl.ds(pl.multiple_of(j * 128 + l * 16, 16), LANES)
                    sb[o] = ib[j, 0, pl.ds(l * 16, LANES)]
                    rb[o] = ib[j, 1, pl.ds(l * 16, LANES)]

    def fire_gath(i, u):
        u = u % 4
        b = u % 2
        k = chunk_of(i)

        @pl.when(k < NCHUNKS)
        def _():
            pltpu.async_copy(pos_hbm.at[sbufs[u]], psbs[b], sem_g[b])
            pltpu.async_copy(pos_hbm.at[rbufs[b]], prbs[b], sem_g[b])

    def wait_gath(i, u):
        u = u % 4
        b = u % 2
        k = chunk_of(i)

        @pl.when(k < NCHUNKS)
        def _():
            dummy = pos_hbm.at[pl.ds(0, CHUNK)]
            pltpu.make_async_copy(dummy, psbs[b], sem_g[b]).wait()
            pltpu.make_async_copy(dummy, prbs[b], sem_g[b]).wait()

    def fire_sct(i, u):
        u = u % 4
        b = u % 2
        k = chunk_of(i)

        @pl.when(k < NCHUNKS)
        def _():
            pltpu.async_copy(ebs[b], accum.at[sbufs[u]], sem_s[b], add=True)

    def wait_sct(i, u):
        b = (u % 4) % 2
        k = chunk_of(i)

        @pl.when((i >= 0) & (k < NCHUNKS))
        def _():
            pltpu.make_async_copy(sx_hbm.at[pl.ds(0, CHUNK)], ebs[b],
                                  sem_s[b]).wait()

    def compute(i, u):
        u = u % 4
        b = u % 2
        k = chunk_of(i)
        psb, prb = psbs[b], prbs[b]
        sxb, syb, szb = sxbs[b], sybs[b], szbs[b]
        eb = ebs[b]

        @pl.when(k < NCHUNKS)
        def _():
            @plsc.parallel_loop(0, CHUNK // LANES, 1, unroll=2)
            def _(g):
                rowids = g * LANES + iota
                eoff = pl.ds(pl.multiple_of(g * LANES, LANES), LANES)
                psx = plsc.load_gather(psb, [rowids, c0])
                psy = plsc.load_gather(psb, [rowids, c1])
                psz = plsc.load_gather(psb, [rowids, c2])
                prx = plsc.load_gather(prb, [rowids, c0])
                pry = plsc.load_gather(prb, [rowids, c1])
                prz = plsc.load_gather(prb, [rowids, c2])
                dx = prx - psx + sxb[eoff]
                dy = pry - psy + syb[eoff]
                dz = prz - psz + szb[eoff]
                r2 = dx * dx + dy * dy + dz * dz
                yi = jnp.int32(0x5F3759DF) - (plsc.bitcast(r2, jnp.int32) >> 1)
                y = plsc.bitcast(yi, jnp.float32)
                h = 0.5 * r2
                for _ in range(3):
                    y = y * (1.5 - (h * y) * y)
                r = r2 * y
                eb[eoff] = 0.5 * jnp.exp(-ALPHA * r)

    fire_lin(0, 0)
    fire_lin(1, 1)
    wait_lin(0, 0)
    deinterleave(0, 0)
    fire_gath(0, 0)

    def _outer(i2, carry):
        for u in range(4):
            i = i2 * 4 + u
            wait_gath(i, u)
            wait_lin(i + 1, u + 1)
            deinterleave(i + 1, u + 1)
            fire_gath(i + 1, u + 1)
            wait_sct(i - 2, u)
            compute(i, u)
            fire_sct(i, u)
            fire_lin(i + 2, u + 2)
        return carry

    lax.fori_loop(0, OUTER, _outer, 0)

    plsc.subcore_barrier()

    cbase = c * N_NODES

    @pl.when(s < NS - 1)
    def _():
        pltpu.sync_copy(accum.at[pl.ds(off, ZSLICE)],
                        out_hbm.at[pl.ds(pl.multiple_of(cbase + off, 8),
                                         ZSLICE)])

    @pl.when(s == NS - 1)
    def _():
        pltpu.sync_copy(accum.at[pl.ds(15 * ZSLICE, ZLAST)],
                        out_hbm.at[pl.ds(pl.multiple_of(cbase + 15 * ZSLICE, 8),
                                         ZLAST)])


_sc_call = pl.kernel(
    _sc_body,
    out_type=jax.ShapeDtypeStruct((NC * N_NODES,), jnp.float32),
    mesh=plsc.VectorSubcoreMesh(core_axis_name="c", subcore_axis_name="s",
                                num_cores=NC, num_subcores=NS),
    compiler_params=pltpu.CompilerParams(needs_layout_passes=False,
                                         use_tc_tiling_on_sc=False),
    scratch_types=[
        tuple(pltpu.VMEM((CHUNK // 128, 2, 128), jnp.int32)
              for _ in range(4)),
        tuple(pltpu.VMEM((CHUNK,), jnp.int32) for _ in range(4)),
        tuple(pltpu.VMEM((CHUNK,), jnp.int32) for _ in range(2)),
        tuple(pltpu.VMEM((CHUNK,), jnp.float32) for _ in range(2)),
        tuple(pltpu.VMEM((CHUNK,), jnp.float32) for _ in range(2)),
        tuple(pltpu.VMEM((CHUNK,), jnp.float32) for _ in range(2)),
        tuple(pltpu.VMEM((CHUNK, 8), jnp.float32) for _ in range(2)),
        tuple(pltpu.VMEM((CHUNK, 8), jnp.float32) for _ in range(2)),
        tuple(pltpu.VMEM((CHUNK,), jnp.float32) for _ in range(2)),
        pltpu.VMEM((ZSLICE,), jnp.float32),
        pltpu.VMEM_SHARED((N_NODES,), jnp.float32),
        tuple(pltpu.SemaphoreType.DMA for _ in range(2)),
        tuple(pltpu.SemaphoreType.DMA for _ in range(2)),
        tuple(pltpu.SemaphoreType.DMA for _ in range(2)),
    ],
)


def _combine_body(part_ref, out_ref):
    out_ref[...] = part_ref[0] + part_ref[1]


_combine = pl.pallas_call(
    _combine_body,
    out_shape=jax.ShapeDtypeStruct((N_NODES,), jnp.float32),
)


def kernel(positions, edge_index, shifts):
    zcol = jnp.zeros((N_NODES,), jnp.float32)
    pos8 = jnp.stack([positions[:, 0], positions[:, 1], positions[:, 2],
                      zcol, zcol, zcol, zcol, zcol], axis=1)
    ei3 = edge_index.reshape(2, N_EDGES // 128, 128).transpose(1, 0, 2)
    partial = _sc_call(pos8, ei3, shifts[:, 0], shifts[:, 1], shifts[:, 2])
    return _combine(partial.reshape(NC, N_NODES))
